# baseline (speedup 1.0000x reference)
import functools

import jax
import jax.numpy as jnp
from jax import lax
from jax.experimental import pallas as pl
from jax.experimental.pallas import tpu as pltpu
from jax.experimental.pallas import tpu_sc as plsc

EMB = 64
S = 2
LOG2S = 1


@functools.lru_cache(maxsize=None)
def _build_sc_kernel(B, NU, NI, C, NW):
    RPW = B // NW
    NCHUNKS = RPW // C
    NB = C // S
    assert B == RPW * NW and RPW == NCHUNKS * C and C == NB * S
    assert NCHUNKS >= 2 and C % 16 == 0

    mesh = plsc.VectorSubcoreMesh(core_axis_name="c", subcore_axis_name="s")

    @functools.partial(
        pl.kernel,
        mesh=mesh,
        compiler_params=pltpu.CompilerParams(
            use_tc_tiling_on_sc=False, needs_layout_passes=False),
        out_type=jax.ShapeDtypeStruct((B, EMB), jnp.float32),
        scratch_types=[
            pltpu.VMEM((C,), jnp.int32),
            pltpu.VMEM((NB, S), jnp.int32),
            pltpu.VMEM((NB, S), jnp.int32),
            pltpu.VMEM((NB, S), jnp.int32),
            pltpu.VMEM((NB, S), jnp.int32),
            pltpu.VMEM((NB, S), jnp.int32),
            pltpu.VMEM((C, EMB), jnp.float32),
            pltpu.VMEM((C, EMB), jnp.float32),
            pltpu.VMEM((S, EMB), jnp.float32),
            pltpu.VMEM((1, EMB), jnp.float32),
            pltpu.SemaphoreType.DMA,
            pltpu.SemaphoreType.DMA,
        ],
    )
    def k(ids_hbm, user_hbm, item_hbm, out_hbm,
          ids_v, ulist, udst, ilist, idst, pdst, bufU, bufI, zbuf, tmp,
          gsem, ssem):
        wid = lax.axis_index("s") * 2 + lax.axis_index("c")
        tile_base = wid * RPW

        zeros16f = jnp.zeros((16,), jnp.float32)
        for r in range(S):
            for q in range(EMB // 16):
                zbuf[r, pl.ds(q * 16, 16)] = zeros16f

        iota16 = lax.broadcasted_iota(jnp.int32, (16,), 0)

        def chunk_body(c, carry):
            base = tile_base + c * C
            tt = jnp.where(c == NCHUNKS - 1, tile_base, base + C)

            pltpu.sync_copy(ids_hbm.at[pl.ds(base, C)], ids_v)

            def grp(g, cnts):
                nu, ni, npd = cnts
                v = ids_v[pl.ds(g * 16, 16)]
                gdst = base + g * 16 + iota16
                mu = (v >= 1) & (v <= NU)
                mi = v > NU
                mp = v == 0
                mu_i = mu.astype(jnp.int32)
                mi_i = mi.astype(jnp.int32)
                mp_i = mp.astype(jnp.int32)
                pu = jnp.maximum(nu + plsc.cumsum(mu_i) - 1, 0)
                pi = jnp.maximum(ni + plsc.cumsum(mi_i) - 1, 0)
                pp = jnp.maximum(npd + plsc.cumsum(mp_i) - 1, 0)
                uidx = jnp.minimum(v - 1, NU - 1)
                iidx = jnp.minimum(v - NU - 1, NI - 1)
                plsc.store_scatter(ulist, [pu >> LOG2S, pu & (S - 1)], uidx, mask=mu)
                plsc.store_scatter(udst, [pu >> LOG2S, pu & (S - 1)], gdst, mask=mu)
                plsc.store_scatter(ilist, [pi >> LOG2S, pi & (S - 1)], iidx, mask=mi)
                plsc.store_scatter(idst, [pi >> LOG2S, pi & (S - 1)], gdst, mask=mi)
                plsc.store_scatter(pdst, [pp >> LOG2S, pp & (S - 1)], gdst, mask=mp)
                return (nu + jnp.sum(mu_i), ni + jnp.sum(mi_i), npd + jnp.sum(mp_i))

            z = jnp.int32(0)
            nu, ni, npd = lax.fori_loop(0, C // 16, grp, (z, z, z))

            def tail_fill(n, list_ref, dst_ref):
                tl = (S - (n & (S - 1))) & (S - 1)
                for h in range((S + 15) // 16):
                    off = h * 16 + iota16
                    m = off < tl
                    pos = jnp.minimum(n + off, C - 1)
                    rc = [pos >> LOG2S, pos & (S - 1)]
                    if list_ref is not None:
                        plsc.store_scatter(list_ref, rc, jnp.zeros((16,), jnp.int32), mask=m)
                    plsc.store_scatter(dst_ref, rc, jnp.broadcast_to(tt, (16,)), mask=m)

            tail_fill(nu, ulist, udst)
            tail_fill(ni, ilist, idst)
            tail_fill(npd, None, pdst)

            nbu = (nu + S - 1) >> LOG2S
            nbi = (ni + S - 1) >> LOG2S
            nbp = (npd + S - 1) >> LOG2S

            def g_u(kb, x):
                pltpu.make_async_copy(user_hbm.at[ulist.at[kb]], bufU.at[pl.ds(kb * S, S)], gsem).start()
                return x

            def g_i(kb, x):
                pltpu.make_async_copy(item_hbm.at[ilist.at[kb]], bufI.at[pl.ds(kb * S, S)], gsem).start()
                return x

            def gw_u(kb, x):
                pltpu.make_async_copy(user_hbm.at[ulist.at[kb]], bufU.at[pl.ds(kb * S, S)], gsem).wait()
                return x

            def gw_i(kb, x):
                pltpu.make_async_copy(item_hbm.at[ilist.at[kb]], bufI.at[pl.ds(kb * S, S)], gsem).wait()
                return x

            lax.fori_loop(0, nbu, g_u, 0)
            lax.fori_loop(0, nbi, g_i, 0)
            lax.fori_loop(0, nbu, gw_u, 0)
            lax.fori_loop(0, nbi, gw_i, 0)

            def s_u(kb, x):
                pltpu.make_async_copy(bufU.at[pl.ds(kb * S, S)], out_hbm.at[udst.at[kb]], ssem).start()
                return x

            def s_i(kb, x):
                pltpu.make_async_copy(bufI.at[pl.ds(kb * S, S)], out_hbm.at[idst.at[kb]], ssem).start()
                return x

            def s_p(kb, x):
                pltpu.make_async_copy(zbuf, out_hbm.at[pdst.at[kb]], ssem).start()
                return x

            def sw_u(kb, x):
                pltpu.make_async_copy(bufU.at[pl.ds(kb * S, S)], out_hbm.at[udst.at[kb]], ssem).wait()
                return x

            def sw_i(kb, x):
                pltpu.make_async_copy(bufI.at[pl.ds(kb * S, S)], out_hbm.at[idst.at[kb]], ssem).wait()
                return x

            def sw_p(kb, x):
                pltpu.make_async_copy(zbuf, out_hbm.at[pdst.at[kb]], ssem).wait()
                return x

            lax.fori_loop(0, nbu, s_u, 0)
            lax.fori_loop(0, nbi, s_i, 0)
            lax.fori_loop(0, nbp, s_p, 0)
            lax.fori_loop(0, nbu, sw_u, 0)
            lax.fori_loop(0, nbi, sw_i, 0)
            lax.fori_loop(0, nbp, sw_p, 0)
            return carry

        lax.fori_loop(0, NCHUNKS, chunk_body, 0)

        pltpu.sync_copy(ids_hbm.at[pl.ds(tile_base, 16)], ids_v.at[pl.ds(0, 16)])
        id0 = ids_v[pl.ds(0, 16)][0]

        @pl.when(id0 == 0)
        def _():
            pltpu.sync_copy(zbuf.at[pl.ds(0, 1)], out_hbm.at[pl.ds(tile_base, 1)])

        @pl.when((id0 >= 1) & (id0 <= NU))
        def _():
            pltpu.sync_copy(user_hbm.at[pl.ds(id0 - 1, 1)], tmp)
            pltpu.sync_copy(tmp, out_hbm.at[pl.ds(tile_base, 1)])

        @pl.when(id0 > NU)
        def _():
            pltpu.sync_copy(item_hbm.at[pl.ds(id0 - NU - 1, 1)], tmp)
            pltpu.sync_copy(tmp, out_hbm.at[pl.ds(tile_base, 1)])

    return k


def kernel(node_ids, user_table, item_table):
    nb, nn = node_ids.shape
    B = nb * nn
    ids = node_ids.reshape(B).astype(jnp.int32)
    NU = int(user_table.shape[0])
    NI = int(item_table.shape[0])
    k = _build_sc_kernel(B, NU, NI, C=800, NW=32)
    out = k(ids, user_table.astype(jnp.float32), item_table.astype(jnp.float32))
    return out.reshape(nb, nn, EMB)


# speedup vs baseline: 1.0440x; 1.0440x over previous
import functools

import jax
import jax.numpy as jnp
from jax import lax
from jax.experimental import pallas as pl
from jax.experimental.pallas import tpu as pltpu
from jax.experimental.pallas import tpu_sc as plsc

EMB = 64
S = 4
LOG2S = 2


@functools.lru_cache(maxsize=None)
def _build_sc_kernel(B, NU, NI, C, NW):
    RPW = B // NW
    NCHUNKS = RPW // C
    NB = C // S
    assert B == RPW * NW and RPW == NCHUNKS * C and C == NB * S
    assert NCHUNKS >= 2 and C % 16 == 0

    mesh = plsc.VectorSubcoreMesh(core_axis_name="c", subcore_axis_name="s")

    @functools.partial(
        pl.kernel,
        mesh=mesh,
        compiler_params=pltpu.CompilerParams(
            use_tc_tiling_on_sc=False, needs_layout_passes=False),
        out_type=jax.ShapeDtypeStruct((B, EMB), jnp.float32),
        scratch_types=[
            pltpu.VMEM((C,), jnp.int32),
            pltpu.VMEM((NB, S), jnp.int32),
            pltpu.VMEM((NB, S), jnp.int32),
            pltpu.VMEM((NB, S), jnp.int32),
            pltpu.VMEM((NB, S), jnp.int32),
            pltpu.VMEM((NB, S), jnp.int32),
            pltpu.VMEM((C, EMB), jnp.float32),
            pltpu.VMEM((C, EMB), jnp.float32),
            pltpu.VMEM((S, EMB), jnp.float32),
            pltpu.VMEM((1, EMB), jnp.float32),
            pltpu.SemaphoreType.DMA,
            pltpu.SemaphoreType.DMA,
        ],
    )
    def k(ids_hbm, user_hbm, item_hbm, out_hbm,
          ids_v, ulist, udst, ilist, idst, pdst, bufU, bufI, zbuf, tmp,
          gsem, ssem):
        wid = lax.axis_index("s") * 2 + lax.axis_index("c")
        tile_base = wid * RPW

        zeros16f = jnp.zeros((16,), jnp.float32)
        for r in range(S):
            for q in range(EMB // 16):
                zbuf[r, pl.ds(q * 16, 16)] = zeros16f

        iota16 = lax.broadcasted_iota(jnp.int32, (16,), 0)

        def chunk_body(c, carry):
            base = tile_base + c * C
            tt = jnp.where(c == NCHUNKS - 1, tile_base, base + C)

            pltpu.sync_copy(ids_hbm.at[pl.ds(base, C)], ids_v)

            def grp(g, cnts):
                nu, ni, npd = cnts
                v = ids_v[pl.ds(g * 16, 16)]
                gdst = base + g * 16 + iota16
                mu = (v >= 1) & (v <= NU)
                mi = v > NU
                mp = v == 0
                mu_i = mu.astype(jnp.int32)
                mi_i = mi.astype(jnp.int32)
                mp_i = mp.astype(jnp.int32)
                pu = jnp.maximum(nu + plsc.cumsum(mu_i) - 1, 0)
                pi = jnp.maximum(ni + plsc.cumsum(mi_i) - 1, 0)
                pp = jnp.maximum(npd + plsc.cumsum(mp_i) - 1, 0)
                uidx = jnp.minimum(v - 1, NU - 1)
                iidx = jnp.minimum(v - NU - 1, NI - 1)
                plsc.store_scatter(ulist, [pu >> LOG2S, pu & (S - 1)], uidx, mask=mu)
                plsc.store_scatter(udst, [pu >> LOG2S, pu & (S - 1)], gdst, mask=mu)
                plsc.store_scatter(ilist, [pi >> LOG2S, pi & (S - 1)], iidx, mask=mi)
                plsc.store_scatter(idst, [pi >> LOG2S, pi & (S - 1)], gdst, mask=mi)
                plsc.store_scatter(pdst, [pp >> LOG2S, pp & (S - 1)], gdst, mask=mp)
                return (nu + jnp.sum(mu_i), ni + jnp.sum(mi_i), npd + jnp.sum(mp_i))

            z = jnp.int32(0)
            nu, ni, npd = lax.fori_loop(0, C // 16, grp, (z, z, z))

            def tail_fill(n, list_ref, dst_ref):
                tl = (S - (n & (S - 1))) & (S - 1)
                for h in range((S + 15) // 16):
                    off = h * 16 + iota16
                    m = off < tl
                    pos = jnp.minimum(n + off, C - 1)
                    rc = [pos >> LOG2S, pos & (S - 1)]
                    if list_ref is not None:
                        plsc.store_scatter(list_ref, rc, jnp.zeros((16,), jnp.int32), mask=m)
                    plsc.store_scatter(dst_ref, rc, jnp.broadcast_to(tt, (16,)), mask=m)

            tail_fill(nu, ulist, udst)
            tail_fill(ni, ilist, idst)
            tail_fill(npd, None, pdst)

            nbu = (nu + S - 1) >> LOG2S
            nbi = (ni + S - 1) >> LOG2S
            nbp = (npd + S - 1) >> LOG2S

            def g_u(kb, x):
                pltpu.make_async_copy(user_hbm.at[ulist.at[kb]], bufU.at[pl.ds(kb * S, S)], gsem).start()
                return x

            def g_i(kb, x):
                pltpu.make_async_copy(item_hbm.at[ilist.at[kb]], bufI.at[pl.ds(kb * S, S)], gsem).start()
                return x

            def gw_u(kb, x):
                pltpu.make_async_copy(user_hbm.at[ulist.at[kb]], bufU.at[pl.ds(kb * S, S)], gsem).wait()
                return x

            def gw_i(kb, x):
                pltpu.make_async_copy(item_hbm.at[ilist.at[kb]], bufI.at[pl.ds(kb * S, S)], gsem).wait()
                return x

            lax.fori_loop(0, nbu, g_u, 0)
            lax.fori_loop(0, nbi, g_i, 0)
            lax.fori_loop(0, nbu, gw_u, 0)
            lax.fori_loop(0, nbi, gw_i, 0)

            def s_u(kb, x):
                pltpu.make_async_copy(bufU.at[pl.ds(kb * S, S)], out_hbm.at[udst.at[kb]], ssem).start()
                return x

            def s_i(kb, x):
                pltpu.make_async_copy(bufI.at[pl.ds(kb * S, S)], out_hbm.at[idst.at[kb]], ssem).start()
                return x

            def s_p(kb, x):
                pltpu.make_async_copy(zbuf, out_hbm.at[pdst.at[kb]], ssem).start()
                return x

            def sw_u(kb, x):
                pltpu.make_async_copy(bufU.at[pl.ds(kb * S, S)], out_hbm.at[udst.at[kb]], ssem).wait()
                return x

            def sw_i(kb, x):
                pltpu.make_async_copy(bufI.at[pl.ds(kb * S, S)], out_hbm.at[idst.at[kb]], ssem).wait()
                return x

            def sw_p(kb, x):
                pltpu.make_async_copy(zbuf, out_hbm.at[pdst.at[kb]], ssem).wait()
                return x

            lax.fori_loop(0, nbu, s_u, 0)
            lax.fori_loop(0, nbi, s_i, 0)
            lax.fori_loop(0, nbp, s_p, 0)
            lax.fori_loop(0, nbu, sw_u, 0)
            lax.fori_loop(0, nbi, sw_i, 0)
            lax.fori_loop(0, nbp, sw_p, 0)
            return carry

        lax.fori_loop(0, NCHUNKS, chunk_body, 0)

        pltpu.sync_copy(ids_hbm.at[pl.ds(tile_base, 16)], ids_v.at[pl.ds(0, 16)])
        id0 = ids_v[pl.ds(0, 16)][0]

        @pl.when(id0 == 0)
        def _():
            pltpu.sync_copy(zbuf.at[pl.ds(0, 1)], out_hbm.at[pl.ds(tile_base, 1)])

        @pl.when((id0 >= 1) & (id0 <= NU))
        def _():
            pltpu.sync_copy(user_hbm.at[pl.ds(id0 - 1, 1)], tmp)
            pltpu.sync_copy(tmp, out_hbm.at[pl.ds(tile_base, 1)])

        @pl.when(id0 > NU)
        def _():
            pltpu.sync_copy(item_hbm.at[pl.ds(id0 - NU - 1, 1)], tmp)
            pltpu.sync_copy(tmp, out_hbm.at[pl.ds(tile_base, 1)])

    return k


def kernel(node_ids, user_table, item_table):
    nb, nn = node_ids.shape
    B = nb * nn
    ids = node_ids.reshape(B).astype(jnp.int32)
    NU = int(user_table.shape[0])
    NI = int(item_table.shape[0])
    k = _build_sc_kernel(B, NU, NI, C=800, NW=32)
    out = k(ids, user_table.astype(jnp.float32), item_table.astype(jnp.float32))
    return out.reshape(nb, nn, EMB)
